# baseline (speedup 1.0000x reference)
import jax
import jax.numpy as jnp
from jax.experimental import pallas as pl
from jax.experimental.pallas import tpu as pltpu


def _prep_kernel(a_ref, deg_ref, a8_ref):
    i = pl.program_id(0)

    @pl.when(i == 0)
    def _():
        deg_ref[...] = jnp.zeros_like(deg_ref)

    a = a_ref[...]
    deg_ref[...] += jnp.sum(a, axis=0, keepdims=True)
    a8_ref[...] = a.astype(jnp.int8)


def _hg_kernel(x_ref, w_ref, deg_ref, h_ref, g_ref):
    h = jnp.dot(x_ref[...], w_ref[...], preferred_element_type=jnp.float32)
    h_ref[...] = h
    d = jax.lax.rsqrt(deg_ref[...] + 1.0)
    g_ref[...] = (h * d).astype(jnp.bfloat16)


def _mm_kernel(bl_r, a8_ref, g_ref, s_ref):
    rt = pl.program_id(1)
    a = a8_ref[...].astype(jnp.bfloat16)
    rows = pl.ds(rt * bl_r, bl_r)
    p = jax.lax.dot_general(
        a, g_ref[rows, :], (((0,), (0,)), ((), ())),
        preferred_element_type=jnp.float32)

    @pl.when(rt == 0)
    def _():
        s_ref[...] = p

    @pl.when(rt > 0)
    def _():
        s_ref[...] += p


def _fin_kernel(s_ref, h_ref, deg_ref, b_ref, out_ref):
    d = jax.lax.rsqrt(deg_ref[...] + 1.0)
    res = d * s_ref[...] + (d * d) * h_ref[...] + b_ref[...]
    out_ref[...] = jnp.maximum(res, 0.0)


@jax.jit
def kernel(x, edge_index, W, b):
    adj = edge_index
    n, d_in = x.shape
    d_out = W.shape[1]

    r1 = min(512, n)
    deg_sum, a8 = pl.pallas_call(
        _prep_kernel,
        grid=(n // r1,),
        in_specs=[pl.BlockSpec((r1, n), lambda i: (i, 0))],
        out_specs=[
            pl.BlockSpec((1, n), lambda i: (0, 0)),
            pl.BlockSpec((r1, n), lambda i: (i, 0)),
        ],
        out_shape=[
            jax.ShapeDtypeStruct((1, n), jnp.float32),
            jax.ShapeDtypeStruct((n, n), jnp.int8),
        ],
    )(adj)

    deg_t = deg_sum.reshape(n, 1)
    b2 = b.reshape(1, d_out)

    rh = min(1024, n)
    h, g = pl.pallas_call(
        _hg_kernel,
        grid=(n // rh,),
        in_specs=[
            pl.BlockSpec((rh, d_in), lambda i: (i, 0)),
            pl.BlockSpec((d_in, d_out), lambda i: (0, 0)),
            pl.BlockSpec((rh, 1), lambda i: (i, 0)),
        ],
        out_specs=[
            pl.BlockSpec((rh, d_out), lambda i: (i, 0)),
            pl.BlockSpec((rh, d_out), lambda i: (i, 0)),
        ],
        out_shape=[
            jax.ShapeDtypeStruct((n, d_out), jnp.float32),
            jax.ShapeDtypeStruct((n, d_out), jnp.bfloat16),
        ],
    )(x, W, deg_t)

    bl_r = min(1024, n)
    bl_c = min(1024, n)

    def mm_body(*refs):
        _mm_kernel(bl_r, *refs)

    s = pl.pallas_call(
        mm_body,
        grid=(n // bl_c, n // bl_r),
        in_specs=[
            pl.BlockSpec((bl_r, bl_c), lambda ct, rt: (rt, ct)),
            pl.BlockSpec((n, d_out), lambda ct, rt: (0, 0)),
        ],
        out_specs=pl.BlockSpec((bl_c, d_out), lambda ct, rt: (ct, 0)),
        out_shape=jax.ShapeDtypeStruct((n, d_out), jnp.float32),
        compiler_params=pltpu.CompilerParams(
            dimension_semantics=("parallel", "arbitrary")),
    )(a8, g)

    out = pl.pallas_call(
        _fin_kernel,
        grid=(n // rh,),
        in_specs=[
            pl.BlockSpec((rh, d_out), lambda i: (i, 0)),
            pl.BlockSpec((rh, d_out), lambda i: (i, 0)),
            pl.BlockSpec((rh, 1), lambda i: (i, 0)),
            pl.BlockSpec((1, d_out), lambda i: (0, 0)),
        ],
        out_specs=pl.BlockSpec((rh, d_out), lambda i: (i, 0)),
        out_shape=jax.ShapeDtypeStruct((n, d_out), jnp.float32),
    )(s, h, deg_t, b2)

    return out


# speedup vs baseline: 1.1391x; 1.1391x over previous
import jax
import jax.numpy as jnp
from jax.experimental import pallas as pl
from jax.experimental.pallas import tpu as pltpu


def _prep_kernel(a_ref, deg_ref, a8_ref):
    i = pl.program_id(0)

    @pl.when(i == 0)
    def _():
        deg_ref[...] = jnp.zeros_like(deg_ref)

    a = a_ref[...]
    deg_ref[...] += jnp.sum(a, axis=0, keepdims=True)
    a8_ref[...] = a.astype(jnp.int8)


def _hg_kernel(x_ref, w_ref, deg_ref, h_ref, g_ref):
    h = jnp.dot(x_ref[...], w_ref[...], preferred_element_type=jnp.float32)
    h_ref[...] = h
    d = jax.lax.rsqrt(deg_ref[...] + 1.0)
    g_ref[...] = (h * d).astype(jnp.bfloat16)


def _mm_kernel(bl_r, a8_ref, g_ref, s_ref):
    rt = pl.program_id(0)
    a = a8_ref[...].astype(jnp.bfloat16)
    rows = pl.ds(rt * bl_r, bl_r)
    p = jax.lax.dot_general(
        a, g_ref[rows, :], (((0,), (0,)), ((), ())),
        preferred_element_type=jnp.float32)

    @pl.when(rt == 0)
    def _():
        s_ref[...] = p

    @pl.when(rt > 0)
    def _():
        s_ref[...] += p


def _fin_kernel(s_ref, h_ref, deg_ref, b_ref, out_ref):
    d = jax.lax.rsqrt(deg_ref[...] + 1.0)
    res = d * s_ref[...] + (d * d) * h_ref[...] + b_ref[...]
    out_ref[...] = jnp.maximum(res, 0.0)


@jax.jit
def kernel(x, edge_index, W, b):
    adj = edge_index
    n, d_in = x.shape
    d_out = W.shape[1]

    r1 = min(512, n)
    deg_sum, a8 = pl.pallas_call(
        _prep_kernel,
        grid=(n // r1,),
        in_specs=[pl.BlockSpec((r1, n), lambda i: (i, 0))],
        out_specs=[
            pl.BlockSpec((1, n), lambda i: (0, 0)),
            pl.BlockSpec((r1, n), lambda i: (i, 0)),
        ],
        out_shape=[
            jax.ShapeDtypeStruct((1, n), jnp.float32),
            jax.ShapeDtypeStruct((n, n), jnp.int8),
        ],
    )(adj)

    deg_t = deg_sum.reshape(n, 1)
    b2 = b.reshape(1, d_out)

    rh = min(1024, n)
    h, g = pl.pallas_call(
        _hg_kernel,
        grid=(n // rh,),
        in_specs=[
            pl.BlockSpec((rh, d_in), lambda i: (i, 0)),
            pl.BlockSpec((d_in, d_out), lambda i: (0, 0)),
            pl.BlockSpec((rh, 1), lambda i: (i, 0)),
        ],
        out_specs=[
            pl.BlockSpec((rh, d_out), lambda i: (i, 0)),
            pl.BlockSpec((rh, d_out), lambda i: (i, 0)),
        ],
        out_shape=[
            jax.ShapeDtypeStruct((n, d_out), jnp.float32),
            jax.ShapeDtypeStruct((n, d_out), jnp.bfloat16),
        ],
    )(x, W, deg_t)

    bl_r = min(1024, n)

    def mm_body(*refs):
        _mm_kernel(bl_r, *refs)

    s = pl.pallas_call(
        mm_body,
        grid=(n // bl_r,),
        in_specs=[
            pl.BlockSpec((bl_r, n), lambda rt: (rt, 0)),
            pl.BlockSpec((n, d_out), lambda rt: (0, 0)),
        ],
        out_specs=pl.BlockSpec((n, d_out), lambda rt: (0, 0)),
        out_shape=jax.ShapeDtypeStruct((n, d_out), jnp.float32),
    )(a8, g)

    out = pl.pallas_call(
        _fin_kernel,
        grid=(n // rh,),
        in_specs=[
            pl.BlockSpec((rh, d_out), lambda i: (i, 0)),
            pl.BlockSpec((rh, d_out), lambda i: (i, 0)),
            pl.BlockSpec((rh, 1), lambda i: (i, 0)),
            pl.BlockSpec((1, d_out), lambda i: (0, 0)),
        ],
        out_specs=pl.BlockSpec((rh, d_out), lambda i: (i, 0)),
        out_shape=jax.ShapeDtypeStruct((n, d_out), jnp.float32),
    )(s, h, deg_t, b2)

    return out


# speedup vs baseline: 1.2432x; 1.0914x over previous
import jax
import jax.numpy as jnp
from jax.experimental import pallas as pl
from jax.experimental.pallas import tpu as pltpu


def _prep_kernel(a_ref, deg_ref, a8_ref):
    i = pl.program_id(0)

    @pl.when(i == 0)
    def _():
        deg_ref[...] = jnp.zeros_like(deg_ref)

    a = a_ref[...]
    deg_ref[...] += jnp.sum(a, axis=0, keepdims=True)
    a8_ref[...] = a.astype(jnp.int8)


def _agg_kernel(bl_r, a8_ref, x_ref, w_ref, deg_ref, b_ref, out_ref,
                h_ref, g_ref):
    rt = pl.program_id(0)
    n_rt = pl.num_programs(0)

    @pl.when(rt == 0)
    def _():
        h = jnp.dot(x_ref[...], w_ref[...],
                    preferred_element_type=jnp.float32)
        h_ref[...] = h
        d = jax.lax.rsqrt(deg_ref[...] + 1.0)
        g_ref[...] = (h * d).astype(jnp.bfloat16)

    a = a8_ref[...].astype(jnp.bfloat16)
    rows = pl.ds(rt * bl_r, bl_r)
    p = jax.lax.dot_general(
        a, g_ref[rows, :], (((0,), (0,)), ((), ())),
        preferred_element_type=jnp.float32)

    @pl.when(rt == 0)
    def _():
        out_ref[...] = p

    @pl.when(rt > 0)
    def _():
        out_ref[...] += p

    @pl.when(rt == n_rt - 1)
    def _():
        d = jax.lax.rsqrt(deg_ref[...] + 1.0)
        res = d * out_ref[...] + (d * d) * h_ref[...] + b_ref[...]
        out_ref[...] = jnp.maximum(res, 0.0)


@jax.jit
def kernel(x, edge_index, W, b):
    adj = edge_index
    n, d_in = x.shape
    d_out = W.shape[1]

    r1 = min(512, n)
    deg_sum, a8 = pl.pallas_call(
        _prep_kernel,
        grid=(n // r1,),
        in_specs=[pl.BlockSpec((r1, n), lambda i: (i, 0))],
        out_specs=[
            pl.BlockSpec((1, n), lambda i: (0, 0)),
            pl.BlockSpec((r1, n), lambda i: (i, 0)),
        ],
        out_shape=[
            jax.ShapeDtypeStruct((1, n), jnp.float32),
            jax.ShapeDtypeStruct((n, n), jnp.int8),
        ],
    )(adj)

    deg_t = deg_sum.reshape(n, 1)
    b2 = b.reshape(1, d_out)

    bl_r = min(1024, n)

    def agg_body(*refs):
        _agg_kernel(bl_r, *refs)

    out = pl.pallas_call(
        agg_body,
        grid=(n // bl_r,),
        in_specs=[
            pl.BlockSpec((bl_r, n), lambda rt: (rt, 0)),
            pl.BlockSpec((n, d_in), lambda rt: (0, 0)),
            pl.BlockSpec((d_in, d_out), lambda rt: (0, 0)),
            pl.BlockSpec((n, 1), lambda rt: (0, 0)),
            pl.BlockSpec((1, d_out), lambda rt: (0, 0)),
        ],
        out_specs=pl.BlockSpec((n, d_out), lambda rt: (0, 0)),
        out_shape=jax.ShapeDtypeStruct((n, d_out), jnp.float32),
        scratch_shapes=[
            pltpu.VMEM((n, d_out), jnp.float32),
            pltpu.VMEM((n, d_out), jnp.bfloat16),
        ],
    )(a8, x, W, deg_t, b2)

    return out


# speedup vs baseline: 1.2492x; 1.0048x over previous
import jax
import jax.numpy as jnp
from jax.experimental import pallas as pl
from jax.experimental.pallas import tpu as pltpu


def _prep_kernel(al_ref, ar_ref, deg_ref, a8l_ref, a8r_ref):
    i = pl.program_id(0)

    @pl.when(i == 0)
    def _():
        deg_ref[...] = jnp.zeros_like(deg_ref)

    al = al_ref[...]
    ar = ar_ref[...]
    half = al.shape[1]
    deg_ref[:, :half] += jnp.sum(al, axis=0, keepdims=True)
    deg_ref[:, half:] += jnp.sum(ar, axis=0, keepdims=True)
    a8l_ref[...] = al.astype(jnp.int8)
    a8r_ref[...] = ar.astype(jnp.int8)


def _agg_kernel(bl_r, nh, a8l_ref, a8r_ref, x_ref, w_ref, deg_ref, b_ref,
                out_ref, h_ref, g_ref):
    rt = pl.program_id(0)
    n_rt = pl.num_programs(0)

    @pl.when(rt == 0)
    def _():
        h = jnp.dot(x_ref[...], w_ref[...],
                    preferred_element_type=jnp.float32)
        h_ref[...] = h
        d = jax.lax.rsqrt(deg_ref[...] + 1.0)
        g_ref[...] = (h * d).astype(jnp.bfloat16)

    rows = pl.ds(rt * bl_r, bl_r)
    gs = g_ref[rows, :]
    dims = (((0,), (0,)), ((), ()))
    pl_ = jax.lax.dot_general(a8l_ref[...].astype(jnp.bfloat16), gs, dims,
                              preferred_element_type=jnp.float32)
    pr_ = jax.lax.dot_general(a8r_ref[...].astype(jnp.bfloat16), gs, dims,
                              preferred_element_type=jnp.float32)

    @pl.when(rt == 0)
    def _():
        out_ref[:nh, :] = pl_
        out_ref[nh:, :] = pr_

    @pl.when(rt > 0)
    def _():
        out_ref[:nh, :] += pl_
        out_ref[nh:, :] += pr_

    @pl.when(rt == n_rt - 1)
    def _():
        d = jax.lax.rsqrt(deg_ref[...] + 1.0)
        res = d * out_ref[...] + (d * d) * h_ref[...] + b_ref[...]
        out_ref[...] = jnp.maximum(res, 0.0)


@jax.jit
def kernel(x, edge_index, W, b):
    adj = edge_index
    n, d_in = x.shape
    d_out = W.shape[1]

    r1 = min(512, n)
    nh = n // 2
    deg_sum, a8l, a8r = pl.pallas_call(
        _prep_kernel,
        grid=(n // r1,),
        in_specs=[
            pl.BlockSpec((r1, nh), lambda i: (i, 0)),
            pl.BlockSpec((r1, nh), lambda i: (i, 1)),
        ],
        out_specs=[
            pl.BlockSpec((1, n), lambda i: (0, 0)),
            pl.BlockSpec((r1, nh), lambda i: (i, 0)),
            pl.BlockSpec((r1, nh), lambda i: (i, 0)),
        ],
        out_shape=[
            jax.ShapeDtypeStruct((1, n), jnp.float32),
            jax.ShapeDtypeStruct((n, nh), jnp.int8),
            jax.ShapeDtypeStruct((n, nh), jnp.int8),
        ],
    )(adj, adj)

    deg_t = deg_sum.reshape(n, 1)
    b2 = b.reshape(1, d_out)

    bl_r = min(1024, n)

    def agg_body(*refs):
        _agg_kernel(bl_r, nh, *refs)

    out = pl.pallas_call(
        agg_body,
        grid=(n // bl_r,),
        in_specs=[
            pl.BlockSpec((bl_r, nh), lambda rt: (rt, 0)),
            pl.BlockSpec((bl_r, nh), lambda rt: (rt, 0)),
            pl.BlockSpec((n, d_in), lambda rt: (0, 0)),
            pl.BlockSpec((d_in, d_out), lambda rt: (0, 0)),
            pl.BlockSpec((n, 1), lambda rt: (0, 0)),
            pl.BlockSpec((1, d_out), lambda rt: (0, 0)),
        ],
        out_specs=pl.BlockSpec((n, d_out), lambda rt: (0, 0)),
        out_shape=jax.ShapeDtypeStruct((n, d_out), jnp.float32),
        scratch_shapes=[
            pltpu.VMEM((n, d_out), jnp.float32),
            pltpu.VMEM((n, d_out), jnp.bfloat16),
        ],
    )(a8l, a8r, x, W, deg_t, b2)

    return out
